# baseline (speedup 1.0000x reference)
import functools

import jax
import jax.numpy as jnp
from jax.experimental import pallas as pl
from jax.experimental.pallas import tpu as pltpu

_VMEM_LIMIT = 48 * 1024 * 1024


def _cparams(*sem):
    return pltpu.CompilerParams(dimension_semantics=sem,
                                vmem_limit_bytes=_VMEM_LIMIT)


def _stem_kernel(x_ref, w_ref, s_ref, b_ref, o_ref):
    xp = x_ref[0]
    acc = None
    for da in range(4):
        for db in range(4):
            tap = xp[da:da + 29, db:db + 112, :].reshape(29 * 112, 12)
            part = jnp.dot(tap, w_ref[da * 4 + db],
                           preferred_element_type=jnp.float32)
            acc = part if acc is None else acc + part
    y = jnp.maximum(acc * s_ref[...] + b_ref[...], 0.0)
    y = y.astype(jnp.bfloat16).reshape(29, 112, 64)
    is_top = pl.program_id(0) % 4 == 0
    rowidx = jax.lax.broadcasted_iota(jnp.int32, (29, 1, 1), 0)
    y = jnp.where((rowidx == 0) & is_top, jnp.bfloat16(0), y)
    y4 = y[0:28].reshape(14, 2, 112, 64)
    e, o = y4[:, 0], y4[:, 1]
    z = y[1:29].reshape(14, 2, 112, 64)[:, 1]
    rm = jnp.maximum(jnp.maximum(e, o), z)
    rm4 = rm.reshape(14, 56, 2, 64)
    ce, co = rm4[:, :, 0], rm4[:, :, 1]
    cz = jnp.zeros((14, 1, 64), jnp.bfloat16)
    cprev = jnp.concatenate([cz, co[:, :-1]], axis=1)
    o_ref[0] = jnp.maximum(jnp.maximum(ce, co), cprev).reshape(784, 64)


def _stem(images, stem_w, stem_scale, stem_bias):
    B = images.shape[0]
    x = images.astype(jnp.bfloat16).reshape(B, 3, 112, 2, 112, 2)
    x = x.transpose(0, 2, 4, 3, 5, 1).reshape(B, 112, 112, 12)
    xp = jnp.pad(x, ((0, 0), (3, 1), (2, 1), (0, 0)))
    xs = jnp.stack([xp[:, 28 * q:28 * q + 32] for q in range(4)], axis=1)
    xs = xs.reshape(B * 4, 32, 115, 12)
    w = stem_w.astype(jnp.bfloat16)
    zero = jnp.zeros((3, 64), jnp.bfloat16)
    taps = []
    for da in range(4):
        for db in range(4):
            chans = []
            for p in range(2):
                for q in range(2):
                    di = 2 * (da - 2) + p + 3
                    dj = 2 * (db - 2) + q + 3
                    ok = 0 <= di < 7 and 0 <= dj < 7
                    chans.append(w[di * 7 + dj] if ok else zero)
            taps.append(jnp.concatenate(chans, axis=0))
    ws2d = jnp.stack(taps)
    out = pl.pallas_call(
        _stem_kernel,
        grid=(B * 4,),
        in_specs=[pl.BlockSpec((1, 32, 115, 12), lambda b: (b, 0, 0, 0)),
                  pl.BlockSpec((16, 12, 64), lambda b: (0, 0, 0)),
                  pl.BlockSpec((1, 64), lambda b: (0, 0)),
                  pl.BlockSpec((1, 64), lambda b: (0, 0))],
        out_specs=pl.BlockSpec((1, 784, 64), lambda b: (b, 0, 0)),
        out_shape=jax.ShapeDtypeStruct((B * 4, 784, 64), jnp.bfloat16),
        compiler_params=_cparams("parallel"),
    )(xs, ws2d, stem_scale, stem_bias)
    return out.reshape(B * 3136, 64)


def _bneck1_kernel(x_ref, w1_ref, s1_ref, b1_ref, w2_ref, s2_ref, b2_ref,
                   w3_ref, s3_ref, b3_ref, *rest, bt, H, W, width, has_down):
    if has_down:
        wd_ref, sd_ref, bd_ref, o_ref, pad_ref = rest
    else:
        o_ref, pad_ref = rest
    HW = H * W
    xb = x_ref[...]
    t1 = jnp.dot(xb, w1_ref[...], preferred_element_type=jnp.float32)
    t1 = jnp.maximum(t1 * s1_ref[...] + b1_ref[...], 0.0).astype(jnp.bfloat16)
    pad_ref[:, 0, :, :] = jnp.zeros((bt, W + 2, width), jnp.bfloat16)
    pad_ref[:, H + 1, :, :] = jnp.zeros((bt, W + 2, width), jnp.bfloat16)
    pad_ref[:, 1:H + 1, 0, :] = jnp.zeros((bt, H, width), jnp.bfloat16)
    pad_ref[:, 1:H + 1, W + 1, :] = jnp.zeros((bt, H, width), jnp.bfloat16)
    pad_ref[:, 1:H + 1, 1:W + 1, :] = t1.reshape(bt, H, W, width)
    padded = pad_ref[...]
    acc = None
    for di in range(3):
        for dj in range(3):
            tap = padded[:, di:di + H, dj:dj + W, :].reshape(bt * HW, width)
            part = jnp.dot(tap, w2_ref[di * 3 + dj],
                           preferred_element_type=jnp.float32)
            acc = part if acc is None else acc + part
    t2 = jnp.maximum(acc * s2_ref[...] + b2_ref[...], 0.0).astype(jnp.bfloat16)
    out = jnp.dot(t2, w3_ref[...], preferred_element_type=jnp.float32)
    out = out * s3_ref[...] + b3_ref[...]
    if has_down:
        ident = jnp.dot(xb, wd_ref[...], preferred_element_type=jnp.float32)
        ident = ident * sd_ref[...] + bd_ref[...]
    else:
        ident = xb.astype(jnp.float32)
    o_ref[...] = jnp.maximum(out + ident, 0.0).astype(o_ref.dtype)


def _bneck1(x, p, *, H, W, bt):
    M, Cin = x.shape
    HW = H * W
    p1, p2, p3 = p["conv1"], p["conv2"], p["conv3"]
    width = p1["w"].shape[-1]
    Cout = p3["w"].shape[-1]
    has_down = "down" in p
    rows = bt * HW

    args = [x, p1["w"][0], p1["scale"], p1["bias"],
            p2["w"], p2["scale"], p2["bias"],
            p3["w"][0], p3["scale"], p3["bias"]]
    in_specs = [pl.BlockSpec((rows, Cin), lambda i: (i, 0)),
                pl.BlockSpec((Cin, width), lambda i: (0, 0)),
                pl.BlockSpec((1, width), lambda i: (0, 0)),
                pl.BlockSpec((1, width), lambda i: (0, 0)),
                pl.BlockSpec((9, width, width), lambda i: (0, 0, 0)),
                pl.BlockSpec((1, width), lambda i: (0, 0)),
                pl.BlockSpec((1, width), lambda i: (0, 0)),
                pl.BlockSpec((width, Cout), lambda i: (0, 0)),
                pl.BlockSpec((1, Cout), lambda i: (0, 0)),
                pl.BlockSpec((1, Cout), lambda i: (0, 0))]
    if has_down:
        pd = p["down"]
        args += [pd["w"][0], pd["scale"], pd["bias"]]
        in_specs += [pl.BlockSpec((Cin, Cout), lambda i: (0, 0)),
                     pl.BlockSpec((1, Cout), lambda i: (0, 0)),
                     pl.BlockSpec((1, Cout), lambda i: (0, 0))]

    return pl.pallas_call(
        functools.partial(_bneck1_kernel, bt=bt, H=H, W=W, width=width,
                          has_down=has_down),
        grid=(M // rows,),
        in_specs=in_specs,
        out_specs=pl.BlockSpec((rows, Cout), lambda i: (i, 0)),
        out_shape=jax.ShapeDtypeStruct((M, Cout), jnp.bfloat16),
        scratch_shapes=[pltpu.VMEM((bt, H + 2, W + 2, width), jnp.bfloat16)],
        compiler_params=_cparams("parallel"),
    )(*args)


def _bneck2_kernel(x_ref, w1_ref, s1_ref, b1_ref, w2_ref, s2_ref, b2_ref,
                   w3_ref, s3_ref, b3_ref, wd_ref, sd_ref, bd_ref,
                   o_ref, pad_ref, *, bt, H, W, width):
    Ho, Wo = H // 2, W // 2
    Mo = bt * Ho * Wo
    Cin = x_ref.shape[-1]
    xb = x_ref[...]
    t1 = jnp.dot(xb, w1_ref[...], preferred_element_type=jnp.float32)
    t1 = jnp.maximum(t1 * s1_ref[...] + b1_ref[...], 0.0).astype(jnp.bfloat16)
    pad_ref[:, 0, :, :] = jnp.zeros((bt, W + 2, width), jnp.bfloat16)
    pad_ref[:, 1:H + 1, 0, :] = jnp.zeros((bt, H, width), jnp.bfloat16)
    pad_ref[:, 1:H + 1, 1:W + 1, :] = t1.reshape(bt, H, W, width)
    y6 = pad_ref[...].reshape(bt, (H + 2) // 2, 2, (W + 2) // 2, 2, width)
    acc = None
    for di in range(3):
        p, r0 = (di % 2, di // 2)
        for dj in range(3):
            q, c0 = (dj % 2, dj // 2)
            tap = y6[:, r0:r0 + Ho, p, c0:c0 + Wo, q, :]
            part = jnp.dot(tap.reshape(Mo, width), w2_ref[di * 3 + dj],
                           preferred_element_type=jnp.float32)
            acc = part if acc is None else acc + part
    t2 = jnp.maximum(acc * s2_ref[...] + b2_ref[...], 0.0).astype(jnp.bfloat16)
    out = jnp.dot(t2, w3_ref[...], preferred_element_type=jnp.float32)
    out = out * s3_ref[...] + b3_ref[...]
    xs = xb.reshape(bt, H // 2, 2, W // 2, 2, Cin)[:, :, 0, :, 0, :]
    xs = xs.reshape(Mo, Cin)
    ident = jnp.dot(xs, wd_ref[...], preferred_element_type=jnp.float32)
    ident = ident * sd_ref[...] + bd_ref[...]
    o_ref[...] = jnp.maximum(out + ident, 0.0).astype(o_ref.dtype)


def _bneck2(x, p, *, H, W, bt):
    M, Cin = x.shape
    B = M // (H * W)
    Ho, Wo = H // 2, W // 2
    p1, p2, p3, pd = p["conv1"], p["conv2"], p["conv3"], p["down"]
    width = p1["w"].shape[-1]
    Cout = p3["w"].shape[-1]
    rows_in = bt * H * W
    rows_out = bt * Ho * Wo

    in_specs = [pl.BlockSpec((rows_in, Cin), lambda i: (i, 0)),
                pl.BlockSpec((Cin, width), lambda i: (0, 0)),
                pl.BlockSpec((1, width), lambda i: (0, 0)),
                pl.BlockSpec((1, width), lambda i: (0, 0)),
                pl.BlockSpec((9, width, width), lambda i: (0, 0, 0)),
                pl.BlockSpec((1, width), lambda i: (0, 0)),
                pl.BlockSpec((1, width), lambda i: (0, 0)),
                pl.BlockSpec((width, Cout), lambda i: (0, 0)),
                pl.BlockSpec((1, Cout), lambda i: (0, 0)),
                pl.BlockSpec((1, Cout), lambda i: (0, 0)),
                pl.BlockSpec((Cin, Cout), lambda i: (0, 0)),
                pl.BlockSpec((1, Cout), lambda i: (0, 0)),
                pl.BlockSpec((1, Cout), lambda i: (0, 0))]
    return pl.pallas_call(
        functools.partial(_bneck2_kernel, bt=bt, H=H, W=W, width=width),
        grid=(B // bt,),
        in_specs=in_specs,
        out_specs=pl.BlockSpec((rows_out, Cout), lambda i: (i, 0)),
        out_shape=jax.ShapeDtypeStruct((B * Ho * Wo, Cout), jnp.bfloat16),
        scratch_shapes=[pltpu.VMEM((bt, H + 2, W + 2, width), jnp.bfloat16)],
        compiler_params=_cparams("parallel"),
    )(x, p1["w"][0], p1["scale"], p1["bias"],
      p2["w"], p2["scale"], p2["bias"],
      p3["w"][0], p3["scale"], p3["bias"],
      pd["w"][0], pd["scale"], pd["bias"])


def _embed_kernel(f_ref, w_ref, b_ref, o_ref, *, B, HW):
    f = f_ref[...].astype(jnp.float32).reshape(B, HW, f_ref.shape[-1])
    pooled = jnp.mean(f, axis=1).astype(jnp.bfloat16)
    o_ref[...] = jnp.dot(pooled, w_ref[...],
                         preferred_element_type=jnp.float32) + b_ref[...]


def _pool_embed(x, embed_w, embed_b, *, B, HW):
    M, C = x.shape
    E = embed_w.shape[1]
    te = 128 if E % 128 == 0 else E
    return pl.pallas_call(
        functools.partial(_embed_kernel, B=B, HW=HW),
        grid=(E // te,),
        in_specs=[pl.BlockSpec((M, C), lambda j: (0, 0)),
                  pl.BlockSpec((C, te), lambda j: (0, j)),
                  pl.BlockSpec((1, te), lambda j: (0, j))],
        out_specs=pl.BlockSpec((B, te), lambda j: (0, j)),
        out_shape=jax.ShapeDtypeStruct((B, E), jnp.float32),
        compiler_params=_cparams("parallel"),
    )(x, embed_w, embed_b)


def kernel(images, stem_w, stem_scale, stem_bias, L0_B0_conv1_w, L0_B0_conv1_scale, L0_B0_conv1_bias, L0_B0_conv2_w, L0_B0_conv2_scale, L0_B0_conv2_bias, L0_B0_conv3_w, L0_B0_conv3_scale, L0_B0_conv3_bias, L0_B0_down_w, L0_B0_down_scale, L0_B0_down_bias, L0_B1_conv1_w, L0_B1_conv1_scale, L0_B1_conv1_bias, L0_B1_conv2_w, L0_B1_conv2_scale, L0_B1_conv2_bias, L0_B1_conv3_w, L0_B1_conv3_scale, L0_B1_conv3_bias, L0_B2_conv1_w, L0_B2_conv1_scale, L0_B2_conv1_bias, L0_B2_conv2_w, L0_B2_conv2_scale, L0_B2_conv2_bias, L0_B2_conv3_w, L0_B2_conv3_scale, L0_B2_conv3_bias, L1_B0_conv1_w, L1_B0_conv1_scale, L1_B0_conv1_bias, L1_B0_conv2_w, L1_B0_conv2_scale, L1_B0_conv2_bias, L1_B0_conv3_w, L1_B0_conv3_scale, L1_B0_conv3_bias, L1_B0_down_w, L1_B0_down_scale, L1_B0_down_bias, L1_B1_conv1_w, L1_B1_conv1_scale, L1_B1_conv1_bias, L1_B1_conv2_w, L1_B1_conv2_scale, L1_B1_conv2_bias, L1_B1_conv3_w, L1_B1_conv3_scale, L1_B1_conv3_bias, L1_B2_conv1_w, L1_B2_conv1_scale, L1_B2_conv1_bias, L1_B2_conv2_w, L1_B2_conv2_scale, L1_B2_conv2_bias, L1_B2_conv3_w, L1_B2_conv3_scale, L1_B2_conv3_bias, L1_B3_conv1_w, L1_B3_conv1_scale, L1_B3_conv1_bias, L1_B3_conv2_w, L1_B3_conv2_scale, L1_B3_conv2_bias, L1_B3_conv3_w, L1_B3_conv3_scale, L1_B3_conv3_bias, L2_B0_conv1_w, L2_B0_conv1_scale, L2_B0_conv1_bias, L2_B0_conv2_w, L2_B0_conv2_scale, L2_B0_conv2_bias, L2_B0_conv3_w, L2_B0_conv3_scale, L2_B0_conv3_bias, L2_B0_down_w, L2_B0_down_scale, L2_B0_down_bias, L2_B1_conv1_w, L2_B1_conv1_scale, L2_B1_conv1_bias, L2_B1_conv2_w, L2_B1_conv2_scale, L2_B1_conv2_bias, L2_B1_conv3_w, L2_B1_conv3_scale, L2_B1_conv3_bias, L2_B2_conv1_w, L2_B2_conv1_scale, L2_B2_conv1_bias, L2_B2_conv2_w, L2_B2_conv2_scale, L2_B2_conv2_bias, L2_B2_conv3_w, L2_B2_conv3_scale, L2_B2_conv3_bias, L2_B3_conv1_w, L2_B3_conv1_scale, L2_B3_conv1_bias, L2_B3_conv2_w, L2_B3_conv2_scale, L2_B3_conv2_bias, L2_B3_conv3_w, L2_B3_conv3_scale, L2_B3_conv3_bias, L2_B4_conv1_w, L2_B4_conv1_scale, L2_B4_conv1_bias, L2_B4_conv2_w, L2_B4_conv2_scale, L2_B4_conv2_bias, L2_B4_conv3_w, L2_B4_conv3_scale, L2_B4_conv3_bias, L2_B5_conv1_w, L2_B5_conv1_scale, L2_B5_conv1_bias, L2_B5_conv2_w, L2_B5_conv2_scale, L2_B5_conv2_bias, L2_B5_conv3_w, L2_B5_conv3_scale, L2_B5_conv3_bias, L3_B0_conv1_w, L3_B0_conv1_scale, L3_B0_conv1_bias, L3_B0_conv2_w, L3_B0_conv2_scale, L3_B0_conv2_bias, L3_B0_conv3_w, L3_B0_conv3_scale, L3_B0_conv3_bias, L3_B0_down_w, L3_B0_down_scale, L3_B0_down_bias, L3_B1_conv1_w, L3_B1_conv1_scale, L3_B1_conv1_bias, L3_B1_conv2_w, L3_B1_conv2_scale, L3_B1_conv2_bias, L3_B1_conv3_w, L3_B1_conv3_scale, L3_B1_conv3_bias, L3_B2_conv1_w, L3_B2_conv1_scale, L3_B2_conv1_bias, L3_B2_conv2_w, L3_B2_conv2_scale, L3_B2_conv2_bias, L3_B2_conv3_w, L3_B2_conv3_scale, L3_B2_conv3_bias, embed_w, embed_b):
    ns = locals()

    def blk(li, bi, stride):
        d = {}
        for cname in ("conv1", "conv2", "conv3", "down"):
            key = "L%d_B%d_%s_w" % (li, bi, cname)
            if key in ns:
                d[cname] = {"w": ns[key],
                            "scale": ns["L%d_B%d_%s_scale" % (li, bi, cname)],
                            "bias": ns["L%d_B%d_%s_bias" % (li, bi, cname)]}
        d["stride"] = stride
        return d

    B = images.shape[0]
    x = _stem(images, stem_w, stem_scale, stem_bias)

    x = _bneck1(x, blk(0, 0, 1), H=56, W=56, bt=1)
    x = _bneck1(x, blk(0, 1, 1), H=56, W=56, bt=1)
    x = _bneck1(x, blk(0, 2, 1), H=56, W=56, bt=1)

    x = _bneck2(x, blk(1, 0, 2), H=56, W=56, bt=2)
    x = _bneck1(x, blk(1, 1, 1), H=28, W=28, bt=2)
    x = _bneck1(x, blk(1, 2, 1), H=28, W=28, bt=2)
    x = _bneck1(x, blk(1, 3, 1), H=28, W=28, bt=2)

    x = _bneck2(x, blk(2, 0, 2), H=28, W=28, bt=4)
    x = _bneck1(x, blk(2, 1, 1), H=14, W=14, bt=4)
    x = _bneck1(x, blk(2, 2, 1), H=14, W=14, bt=4)
    x = _bneck1(x, blk(2, 3, 1), H=14, W=14, bt=4)
    x = _bneck1(x, blk(2, 4, 1), H=14, W=14, bt=4)
    x = _bneck1(x, blk(2, 5, 1), H=14, W=14, bt=4)

    x = _bneck2(x, blk(3, 0, 2), H=14, W=14, bt=8)
    x = _bneck1(x, blk(3, 1, 1), H=7, W=7, bt=8)
    x = _bneck1(x, blk(3, 2, 1), H=7, W=7, bt=8)

    return _pool_embed(x, embed_w, embed_b, B=B, HW=49)


# speedup vs baseline: 2.1199x; 2.1199x over previous
import functools

import jax
import jax.numpy as jnp
from jax.experimental import pallas as pl
from jax.experimental.pallas import tpu as pltpu

_VMEM_LIMIT = 48 * 1024 * 1024


def _cparams(*sem):
    return pltpu.CompilerParams(dimension_semantics=sem,
                                vmem_limit_bytes=_VMEM_LIMIT)


def _stem_kernel(x_ref, w_ref, s_ref, b_ref, o_ref):
    xp = x_ref[0]
    acc = None
    for da in range(4):
        for db in range(4):
            tap = xp[da:da + 29, db:db + 112, :].reshape(29 * 112, 12)
            part = jnp.dot(tap, w_ref[da * 4 + db],
                           preferred_element_type=jnp.float32)
            acc = part if acc is None else acc + part
    y = jnp.maximum(acc * s_ref[...] + b_ref[...], 0.0)
    y = y.astype(jnp.bfloat16).reshape(29, 112, 64)
    is_top = pl.program_id(0) % 4 == 0
    rowidx = jax.lax.broadcasted_iota(jnp.int32, (29, 1, 1), 0)
    y = jnp.where((rowidx == 0) & is_top, jnp.bfloat16(0), y)
    y4 = y[0:28].reshape(14, 2, 112, 64)
    e, o = y4[:, 0], y4[:, 1]
    z = y[1:29].reshape(14, 2, 112, 64)[:, 1]
    rm = jnp.maximum(jnp.maximum(e, o), z)
    rm4 = rm.reshape(14, 56, 2, 64)
    ce, co = rm4[:, :, 0], rm4[:, :, 1]
    cz = jnp.zeros((14, 1, 64), jnp.bfloat16)
    cprev = jnp.concatenate([cz, co[:, :-1]], axis=1)
    o_ref[0] = jnp.maximum(jnp.maximum(ce, co), cprev).reshape(784, 64)


def _s2d_kernel(x_ref, t_ref, o_ref):
    xi = x_ref[0].astype(jnp.bfloat16).reshape(3, 112, 2, 224)
    acc = None
    for c in range(3):
        for p in range(2):
            part = jnp.dot(xi[c, :, p, :], t_ref[c * 2 + p],
                           preferred_element_type=jnp.float32)
            acc = part if acc is None else acc + part
    y = acc.astype(jnp.bfloat16)
    z3 = jnp.zeros((3, 1380), jnp.bfloat16)
    z1 = jnp.zeros((1, 1380), jnp.bfloat16)
    yp = jnp.concatenate([z3, y, z1], axis=0)
    for q in range(4):
        o_ref[0, q] = yp[28 * q:28 * q + 32]


def _stem(images, stem_w, stem_scale, stem_bias):
    B = images.shape[0]
    col = jnp.arange(224)[:, None]
    bb = jnp.arange(115)[None, :]
    m = [(col == 2 * bb - 4 + q).astype(jnp.bfloat16) for q in range(2)]
    ts = []
    for c in range(3):
        for p in range(2):
            t = jnp.zeros((224, 115, 12), jnp.bfloat16)
            t = t.at[:, :, p * 6 + c].set(m[0])
            t = t.at[:, :, p * 6 + 3 + c].set(m[1])
            ts.append(t.reshape(224, 1380))
    ts = jnp.stack(ts)
    xq = pl.pallas_call(
        _s2d_kernel,
        grid=(B,),
        in_specs=[pl.BlockSpec((1, 3, 224, 224), lambda b: (b, 0, 0, 0)),
                  pl.BlockSpec((6, 224, 1380), lambda b: (0, 0, 0))],
        out_specs=pl.BlockSpec((1, 4, 32, 1380), lambda b: (b, 0, 0, 0)),
        out_shape=jax.ShapeDtypeStruct((B, 4, 32, 1380), jnp.bfloat16),
        compiler_params=_cparams("parallel"),
    )(images, ts)
    xs = xq.reshape(B * 4, 32, 115, 12)
    w = stem_w.astype(jnp.bfloat16)
    zero = jnp.zeros((3, 64), jnp.bfloat16)
    taps = []
    for da in range(4):
        for db in range(4):
            chans = []
            for p in range(2):
                for q in range(2):
                    di = 2 * (da - 2) + p + 3
                    dj = 2 * (db - 2) + q + 3
                    ok = 0 <= di < 7 and 0 <= dj < 7
                    chans.append(w[di * 7 + dj] if ok else zero)
            taps.append(jnp.concatenate(chans, axis=0))
    ws2d = jnp.stack(taps)
    out = pl.pallas_call(
        _stem_kernel,
        grid=(B * 4,),
        in_specs=[pl.BlockSpec((1, 32, 115, 12), lambda b: (b, 0, 0, 0)),
                  pl.BlockSpec((16, 12, 64), lambda b: (0, 0, 0)),
                  pl.BlockSpec((1, 64), lambda b: (0, 0)),
                  pl.BlockSpec((1, 64), lambda b: (0, 0))],
        out_specs=pl.BlockSpec((1, 784, 64), lambda b: (b, 0, 0)),
        out_shape=jax.ShapeDtypeStruct((B * 4, 784, 64), jnp.bfloat16),
        compiler_params=_cparams("parallel"),
    )(xs, ws2d, stem_scale, stem_bias)
    return out.reshape(B * 3136, 64)


def _bneck1_kernel(x_ref, w1_ref, s1_ref, b1_ref, w2_ref, s2_ref, b2_ref,
                   w3_ref, s3_ref, b3_ref, *rest, bt, H, W, width, has_down):
    if has_down:
        wd_ref, sd_ref, bd_ref, o_ref, pad_ref = rest
    else:
        o_ref, pad_ref = rest
    HW = H * W
    xb = x_ref[...]
    t1 = jnp.dot(xb, w1_ref[...], preferred_element_type=jnp.float32)
    t1 = jnp.maximum(t1 * s1_ref[...] + b1_ref[...], 0.0).astype(jnp.bfloat16)
    pad_ref[:, 0, :, :] = jnp.zeros((bt, W + 2, width), jnp.bfloat16)
    pad_ref[:, H + 1, :, :] = jnp.zeros((bt, W + 2, width), jnp.bfloat16)
    pad_ref[:, 1:H + 1, 0, :] = jnp.zeros((bt, H, width), jnp.bfloat16)
    pad_ref[:, 1:H + 1, W + 1, :] = jnp.zeros((bt, H, width), jnp.bfloat16)
    pad_ref[:, 1:H + 1, 1:W + 1, :] = t1.reshape(bt, H, W, width)
    padded = pad_ref[...]
    acc = None
    for di in range(3):
        for dj in range(3):
            tap = padded[:, di:di + H, dj:dj + W, :].reshape(bt * HW, width)
            part = jnp.dot(tap, w2_ref[di * 3 + dj],
                           preferred_element_type=jnp.float32)
            acc = part if acc is None else acc + part
    t2 = jnp.maximum(acc * s2_ref[...] + b2_ref[...], 0.0).astype(jnp.bfloat16)
    out = jnp.dot(t2, w3_ref[...], preferred_element_type=jnp.float32)
    out = out * s3_ref[...] + b3_ref[...]
    if has_down:
        ident = jnp.dot(xb, wd_ref[...], preferred_element_type=jnp.float32)
        ident = ident * sd_ref[...] + bd_ref[...]
    else:
        ident = xb.astype(jnp.float32)
    o_ref[...] = jnp.maximum(out + ident, 0.0).astype(o_ref.dtype)


def _bneck1(x, p, *, H, W, bt):
    M, Cin = x.shape
    HW = H * W
    p1, p2, p3 = p["conv1"], p["conv2"], p["conv3"]
    width = p1["w"].shape[-1]
    Cout = p3["w"].shape[-1]
    has_down = "down" in p
    rows = bt * HW

    args = [x, p1["w"][0], p1["scale"], p1["bias"],
            p2["w"], p2["scale"], p2["bias"],
            p3["w"][0], p3["scale"], p3["bias"]]
    in_specs = [pl.BlockSpec((rows, Cin), lambda i: (i, 0)),
                pl.BlockSpec((Cin, width), lambda i: (0, 0)),
                pl.BlockSpec((1, width), lambda i: (0, 0)),
                pl.BlockSpec((1, width), lambda i: (0, 0)),
                pl.BlockSpec((9, width, width), lambda i: (0, 0, 0)),
                pl.BlockSpec((1, width), lambda i: (0, 0)),
                pl.BlockSpec((1, width), lambda i: (0, 0)),
                pl.BlockSpec((width, Cout), lambda i: (0, 0)),
                pl.BlockSpec((1, Cout), lambda i: (0, 0)),
                pl.BlockSpec((1, Cout), lambda i: (0, 0))]
    if has_down:
        pd = p["down"]
        args += [pd["w"][0], pd["scale"], pd["bias"]]
        in_specs += [pl.BlockSpec((Cin, Cout), lambda i: (0, 0)),
                     pl.BlockSpec((1, Cout), lambda i: (0, 0)),
                     pl.BlockSpec((1, Cout), lambda i: (0, 0))]

    return pl.pallas_call(
        functools.partial(_bneck1_kernel, bt=bt, H=H, W=W, width=width,
                          has_down=has_down),
        grid=(M // rows,),
        in_specs=in_specs,
        out_specs=pl.BlockSpec((rows, Cout), lambda i: (i, 0)),
        out_shape=jax.ShapeDtypeStruct((M, Cout), jnp.bfloat16),
        scratch_shapes=[pltpu.VMEM((bt, H + 2, W + 2, width), jnp.bfloat16)],
        compiler_params=_cparams("parallel"),
    )(*args)


def _bneck2_kernel(x_ref, w1_ref, s1_ref, b1_ref, w2_ref, s2_ref, b2_ref,
                   w3_ref, s3_ref, b3_ref, wd_ref, sd_ref, bd_ref,
                   o_ref, pad_ref, *, bt, H, W, width):
    Ho, Wo = H // 2, W // 2
    Mo = bt * Ho * Wo
    Cin = x_ref.shape[-1]
    xb = x_ref[...]
    t1 = jnp.dot(xb, w1_ref[...], preferred_element_type=jnp.float32)
    t1 = jnp.maximum(t1 * s1_ref[...] + b1_ref[...], 0.0).astype(jnp.bfloat16)
    pad_ref[:, 0, :, :] = jnp.zeros((bt, W + 2, width), jnp.bfloat16)
    pad_ref[:, 1:H + 1, 0, :] = jnp.zeros((bt, H, width), jnp.bfloat16)
    pad_ref[:, 1:H + 1, 1:W + 1, :] = t1.reshape(bt, H, W, width)
    y6 = pad_ref[...].reshape(bt, (H + 2) // 2, 2, (W + 2) // 2, 2, width)
    acc = None
    for di in range(3):
        p, r0 = (di % 2, di // 2)
        for dj in range(3):
            q, c0 = (dj % 2, dj // 2)
            tap = y6[:, r0:r0 + Ho, p, c0:c0 + Wo, q, :]
            part = jnp.dot(tap.reshape(Mo, width), w2_ref[di * 3 + dj],
                           preferred_element_type=jnp.float32)
            acc = part if acc is None else acc + part
    t2 = jnp.maximum(acc * s2_ref[...] + b2_ref[...], 0.0).astype(jnp.bfloat16)
    out = jnp.dot(t2, w3_ref[...], preferred_element_type=jnp.float32)
    out = out * s3_ref[...] + b3_ref[...]
    xs = xb.reshape(bt, H // 2, 2, W // 2, 2, Cin)[:, :, 0, :, 0, :]
    xs = xs.reshape(Mo, Cin)
    ident = jnp.dot(xs, wd_ref[...], preferred_element_type=jnp.float32)
    ident = ident * sd_ref[...] + bd_ref[...]
    o_ref[...] = jnp.maximum(out + ident, 0.0).astype(o_ref.dtype)


def _bneck2(x, p, *, H, W, bt):
    M, Cin = x.shape
    B = M // (H * W)
    Ho, Wo = H // 2, W // 2
    p1, p2, p3, pd = p["conv1"], p["conv2"], p["conv3"], p["down"]
    width = p1["w"].shape[-1]
    Cout = p3["w"].shape[-1]
    rows_in = bt * H * W
    rows_out = bt * Ho * Wo

    in_specs = [pl.BlockSpec((rows_in, Cin), lambda i: (i, 0)),
                pl.BlockSpec((Cin, width), lambda i: (0, 0)),
                pl.BlockSpec((1, width), lambda i: (0, 0)),
                pl.BlockSpec((1, width), lambda i: (0, 0)),
                pl.BlockSpec((9, width, width), lambda i: (0, 0, 0)),
                pl.BlockSpec((1, width), lambda i: (0, 0)),
                pl.BlockSpec((1, width), lambda i: (0, 0)),
                pl.BlockSpec((width, Cout), lambda i: (0, 0)),
                pl.BlockSpec((1, Cout), lambda i: (0, 0)),
                pl.BlockSpec((1, Cout), lambda i: (0, 0)),
                pl.BlockSpec((Cin, Cout), lambda i: (0, 0)),
                pl.BlockSpec((1, Cout), lambda i: (0, 0)),
                pl.BlockSpec((1, Cout), lambda i: (0, 0))]
    return pl.pallas_call(
        functools.partial(_bneck2_kernel, bt=bt, H=H, W=W, width=width),
        grid=(B // bt,),
        in_specs=in_specs,
        out_specs=pl.BlockSpec((rows_out, Cout), lambda i: (i, 0)),
        out_shape=jax.ShapeDtypeStruct((B * Ho * Wo, Cout), jnp.bfloat16),
        scratch_shapes=[pltpu.VMEM((bt, H + 2, W + 2, width), jnp.bfloat16)],
        compiler_params=_cparams("parallel"),
    )(x, p1["w"][0], p1["scale"], p1["bias"],
      p2["w"], p2["scale"], p2["bias"],
      p3["w"][0], p3["scale"], p3["bias"],
      pd["w"][0], pd["scale"], pd["bias"])


def _embed_kernel(f_ref, w_ref, b_ref, o_ref, *, B, HW):
    f = f_ref[...].astype(jnp.float32).reshape(B, HW, f_ref.shape[-1])
    pooled = jnp.mean(f, axis=1).astype(jnp.bfloat16)
    o_ref[...] = jnp.dot(pooled, w_ref[...],
                         preferred_element_type=jnp.float32) + b_ref[...]


def _pool_embed(x, embed_w, embed_b, *, B, HW):
    M, C = x.shape
    E = embed_w.shape[1]
    te = 128 if E % 128 == 0 else E
    return pl.pallas_call(
        functools.partial(_embed_kernel, B=B, HW=HW),
        grid=(E // te,),
        in_specs=[pl.BlockSpec((M, C), lambda j: (0, 0)),
                  pl.BlockSpec((C, te), lambda j: (0, j)),
                  pl.BlockSpec((1, te), lambda j: (0, j))],
        out_specs=pl.BlockSpec((B, te), lambda j: (0, j)),
        out_shape=jax.ShapeDtypeStruct((B, E), jnp.float32),
        compiler_params=_cparams("parallel"),
    )(x, embed_w, embed_b)


def kernel(images, stem_w, stem_scale, stem_bias, L0_B0_conv1_w, L0_B0_conv1_scale, L0_B0_conv1_bias, L0_B0_conv2_w, L0_B0_conv2_scale, L0_B0_conv2_bias, L0_B0_conv3_w, L0_B0_conv3_scale, L0_B0_conv3_bias, L0_B0_down_w, L0_B0_down_scale, L0_B0_down_bias, L0_B1_conv1_w, L0_B1_conv1_scale, L0_B1_conv1_bias, L0_B1_conv2_w, L0_B1_conv2_scale, L0_B1_conv2_bias, L0_B1_conv3_w, L0_B1_conv3_scale, L0_B1_conv3_bias, L0_B2_conv1_w, L0_B2_conv1_scale, L0_B2_conv1_bias, L0_B2_conv2_w, L0_B2_conv2_scale, L0_B2_conv2_bias, L0_B2_conv3_w, L0_B2_conv3_scale, L0_B2_conv3_bias, L1_B0_conv1_w, L1_B0_conv1_scale, L1_B0_conv1_bias, L1_B0_conv2_w, L1_B0_conv2_scale, L1_B0_conv2_bias, L1_B0_conv3_w, L1_B0_conv3_scale, L1_B0_conv3_bias, L1_B0_down_w, L1_B0_down_scale, L1_B0_down_bias, L1_B1_conv1_w, L1_B1_conv1_scale, L1_B1_conv1_bias, L1_B1_conv2_w, L1_B1_conv2_scale, L1_B1_conv2_bias, L1_B1_conv3_w, L1_B1_conv3_scale, L1_B1_conv3_bias, L1_B2_conv1_w, L1_B2_conv1_scale, L1_B2_conv1_bias, L1_B2_conv2_w, L1_B2_conv2_scale, L1_B2_conv2_bias, L1_B2_conv3_w, L1_B2_conv3_scale, L1_B2_conv3_bias, L1_B3_conv1_w, L1_B3_conv1_scale, L1_B3_conv1_bias, L1_B3_conv2_w, L1_B3_conv2_scale, L1_B3_conv2_bias, L1_B3_conv3_w, L1_B3_conv3_scale, L1_B3_conv3_bias, L2_B0_conv1_w, L2_B0_conv1_scale, L2_B0_conv1_bias, L2_B0_conv2_w, L2_B0_conv2_scale, L2_B0_conv2_bias, L2_B0_conv3_w, L2_B0_conv3_scale, L2_B0_conv3_bias, L2_B0_down_w, L2_B0_down_scale, L2_B0_down_bias, L2_B1_conv1_w, L2_B1_conv1_scale, L2_B1_conv1_bias, L2_B1_conv2_w, L2_B1_conv2_scale, L2_B1_conv2_bias, L2_B1_conv3_w, L2_B1_conv3_scale, L2_B1_conv3_bias, L2_B2_conv1_w, L2_B2_conv1_scale, L2_B2_conv1_bias, L2_B2_conv2_w, L2_B2_conv2_scale, L2_B2_conv2_bias, L2_B2_conv3_w, L2_B2_conv3_scale, L2_B2_conv3_bias, L2_B3_conv1_w, L2_B3_conv1_scale, L2_B3_conv1_bias, L2_B3_conv2_w, L2_B3_conv2_scale, L2_B3_conv2_bias, L2_B3_conv3_w, L2_B3_conv3_scale, L2_B3_conv3_bias, L2_B4_conv1_w, L2_B4_conv1_scale, L2_B4_conv1_bias, L2_B4_conv2_w, L2_B4_conv2_scale, L2_B4_conv2_bias, L2_B4_conv3_w, L2_B4_conv3_scale, L2_B4_conv3_bias, L2_B5_conv1_w, L2_B5_conv1_scale, L2_B5_conv1_bias, L2_B5_conv2_w, L2_B5_conv2_scale, L2_B5_conv2_bias, L2_B5_conv3_w, L2_B5_conv3_scale, L2_B5_conv3_bias, L3_B0_conv1_w, L3_B0_conv1_scale, L3_B0_conv1_bias, L3_B0_conv2_w, L3_B0_conv2_scale, L3_B0_conv2_bias, L3_B0_conv3_w, L3_B0_conv3_scale, L3_B0_conv3_bias, L3_B0_down_w, L3_B0_down_scale, L3_B0_down_bias, L3_B1_conv1_w, L3_B1_conv1_scale, L3_B1_conv1_bias, L3_B1_conv2_w, L3_B1_conv2_scale, L3_B1_conv2_bias, L3_B1_conv3_w, L3_B1_conv3_scale, L3_B1_conv3_bias, L3_B2_conv1_w, L3_B2_conv1_scale, L3_B2_conv1_bias, L3_B2_conv2_w, L3_B2_conv2_scale, L3_B2_conv2_bias, L3_B2_conv3_w, L3_B2_conv3_scale, L3_B2_conv3_bias, embed_w, embed_b):
    ns = locals()

    def blk(li, bi, stride):
        d = {}
        for cname in ("conv1", "conv2", "conv3", "down"):
            key = "L%d_B%d_%s_w" % (li, bi, cname)
            if key in ns:
                d[cname] = {"w": ns[key],
                            "scale": ns["L%d_B%d_%s_scale" % (li, bi, cname)],
                            "bias": ns["L%d_B%d_%s_bias" % (li, bi, cname)]}
        d["stride"] = stride
        return d

    B = images.shape[0]
    x = _stem(images, stem_w, stem_scale, stem_bias)

    x = _bneck1(x, blk(0, 0, 1), H=56, W=56, bt=1)
    x = _bneck1(x, blk(0, 1, 1), H=56, W=56, bt=1)
    x = _bneck1(x, blk(0, 2, 1), H=56, W=56, bt=1)

    x = _bneck2(x, blk(1, 0, 2), H=56, W=56, bt=2)
    x = _bneck1(x, blk(1, 1, 1), H=28, W=28, bt=2)
    x = _bneck1(x, blk(1, 2, 1), H=28, W=28, bt=2)
    x = _bneck1(x, blk(1, 3, 1), H=28, W=28, bt=2)

    x = _bneck2(x, blk(2, 0, 2), H=28, W=28, bt=4)
    x = _bneck1(x, blk(2, 1, 1), H=14, W=14, bt=4)
    x = _bneck1(x, blk(2, 2, 1), H=14, W=14, bt=4)
    x = _bneck1(x, blk(2, 3, 1), H=14, W=14, bt=4)
    x = _bneck1(x, blk(2, 4, 1), H=14, W=14, bt=4)
    x = _bneck1(x, blk(2, 5, 1), H=14, W=14, bt=4)

    x = _bneck2(x, blk(3, 0, 2), H=14, W=14, bt=8)
    x = _bneck1(x, blk(3, 1, 1), H=7, W=7, bt=8)
    x = _bneck1(x, blk(3, 2, 1), H=7, W=7, bt=8)

    return _pool_embed(x, embed_w, embed_b, B=B, HW=49)


# speedup vs baseline: 2.4362x; 1.1492x over previous
import functools

import jax
import jax.numpy as jnp
from jax.experimental import pallas as pl
from jax.experimental.pallas import tpu as pltpu

_VMEM_LIMIT = 48 * 1024 * 1024


def _cparams(*sem):
    return pltpu.CompilerParams(dimension_semantics=sem,
                                vmem_limit_bytes=_VMEM_LIMIT)


_STEM_SLABS = [(p, s, c)
               for p in range(2)
               for s in (range(-1, 2) if p == 0 else range(-2, 2))
               for c in range(3)]


def _stem_kernel(x_ref, t_ref, s_ref, b_ref, o_ref):
    xi = x_ref[0, 0].astype(jnp.bfloat16).reshape(3, 112, 2, 64)
    z2 = jnp.zeros((2, 64), jnp.bfloat16)
    z1 = jnp.zeros((1, 64), jnp.bfloat16)
    padded = {}
    for c in range(3):
        for p in range(2):
            padded[(p, c)] = jnp.concatenate([z2, xi[c, :, p, :], z1], axis=0)
    acc = None
    for idx, (p, s, c) in enumerate(_STEM_SLABS):
        lhs = padded[(p, c)][s + 2:s + 114]
        part = jnp.dot(lhs, t_ref[idx], preferred_element_type=jnp.float32)
        acc = part if acc is None else acc + part
    y = jnp.maximum(acc * s_ref[...] + b_ref[...], 0.0)
    y = y.astype(jnp.bfloat16)
    y4 = y.reshape(56, 2, 1856)
    e, o = y4[:, 0], y4[:, 1]
    pz = jnp.zeros((1, 1856), jnp.bfloat16)
    prev = jnp.concatenate([pz, o[:-1]], axis=0)
    o_ref[0, 0] = jnp.maximum(jnp.maximum(e, o), prev)


def _colpool_kernel(x_ref, o_ref):
    x = x_ref[0, 0]
    jj = jax.lax.broadcasted_iota(jnp.int32, (1, 29, 1), 1)
    first = pl.program_id(0) == 0
    x = jnp.where((jj == 0) & first, jnp.bfloat16(0), x)
    e4 = x[:, 0:28].reshape(56, 14, 2, 64)
    ce, co = e4[:, :, 0], e4[:, :, 1]
    z = x[:, 1:29].reshape(56, 14, 2, 64)[:, :, 1]
    o_ref[0, :, 0] = jnp.maximum(jnp.maximum(ce, co), z)


def _stem(images, stem_w, stem_scale, stem_bias):
    B = images.shape[0]
    imgp = jnp.pad(images, ((0, 0), (0, 0), (0, 0), (5, 3)))
    img4 = jnp.stack([imgp[:, :, :, 56 * m:56 * m + 64] for m in range(4)],
                     axis=1)
    w = stem_w.astype(jnp.bfloat16)
    k = jnp.arange(64)[:, None]
    jj = jnp.arange(29)[None, :]
    hots = [(k == 2 * jj + dj).astype(jnp.bfloat16) for dj in range(7)]
    tws = []
    for (p, s, c) in _STEM_SLABS:
        di = 2 * s + 3 + p
        t = sum(hots[dj][:, :, None] * w[di * 7 + dj, c][None, None, :]
                for dj in range(7))
        tws.append(t.reshape(64, 1856))
    tw = jnp.stack(tws)
    sc = jnp.tile(stem_scale, (1, 29))
    bi = jnp.tile(stem_bias, (1, 29))
    ym = pl.pallas_call(
        _stem_kernel,
        grid=(4, B),
        in_specs=[pl.BlockSpec((1, 1, 3, 224, 64), lambda m, b: (b, m, 0, 0, 0)),
                  pl.BlockSpec((21, 64, 1856), lambda m, b: (0, 0, 0)),
                  pl.BlockSpec((1, 1856), lambda m, b: (0, 0)),
                  pl.BlockSpec((1, 1856), lambda m, b: (0, 0))],
        out_specs=pl.BlockSpec((1, 1, 56, 1856), lambda m, b: (b, m, 0, 0)),
        out_shape=jax.ShapeDtypeStruct((B, 4, 56, 1856), jnp.bfloat16),
        compiler_params=_cparams("parallel", "parallel"),
    )(img4, tw, sc, bi)
    ym = ym.reshape(B, 4, 56, 29, 64)
    out = pl.pallas_call(
        _colpool_kernel,
        grid=(4, B),
        in_specs=[pl.BlockSpec((1, 1, 56, 29, 64),
                               lambda m, b: (b, m, 0, 0, 0))],
        out_specs=pl.BlockSpec((1, 56, 1, 14, 64),
                               lambda m, b: (b, 0, m, 0, 0)),
        out_shape=jax.ShapeDtypeStruct((B, 56, 4, 14, 64), jnp.bfloat16),
        compiler_params=_cparams("parallel", "parallel"),
    )(ym)
    return out.reshape(B * 3136, 64)


def _bneck1_kernel(x_ref, w1_ref, s1_ref, b1_ref, w2_ref, s2_ref, b2_ref,
                   w3_ref, s3_ref, b3_ref, *rest, bt, H, W, width, has_down):
    if has_down:
        wd_ref, sd_ref, bd_ref, o_ref, pad_ref = rest
    else:
        o_ref, pad_ref = rest
    HW = H * W
    xb = x_ref[...]
    t1 = jnp.dot(xb, w1_ref[...], preferred_element_type=jnp.float32)
    t1 = jnp.maximum(t1 * s1_ref[...] + b1_ref[...], 0.0).astype(jnp.bfloat16)
    pad_ref[:, 0, :, :] = jnp.zeros((bt, W + 2, width), jnp.bfloat16)
    pad_ref[:, H + 1, :, :] = jnp.zeros((bt, W + 2, width), jnp.bfloat16)
    pad_ref[:, 1:H + 1, 0, :] = jnp.zeros((bt, H, width), jnp.bfloat16)
    pad_ref[:, 1:H + 1, W + 1, :] = jnp.zeros((bt, H, width), jnp.bfloat16)
    pad_ref[:, 1:H + 1, 1:W + 1, :] = t1.reshape(bt, H, W, width)
    padded = pad_ref[...]
    acc = None
    for di in range(3):
        for dj in range(3):
            tap = padded[:, di:di + H, dj:dj + W, :].reshape(bt * HW, width)
            part = jnp.dot(tap, w2_ref[di * 3 + dj],
                           preferred_element_type=jnp.float32)
            acc = part if acc is None else acc + part
    t2 = jnp.maximum(acc * s2_ref[...] + b2_ref[...], 0.0).astype(jnp.bfloat16)
    out = jnp.dot(t2, w3_ref[...], preferred_element_type=jnp.float32)
    out = out * s3_ref[...] + b3_ref[...]
    if has_down:
        ident = jnp.dot(xb, wd_ref[...], preferred_element_type=jnp.float32)
        ident = ident * sd_ref[...] + bd_ref[...]
    else:
        ident = xb.astype(jnp.float32)
    o_ref[...] = jnp.maximum(out + ident, 0.0).astype(o_ref.dtype)


def _bneck1(x, p, *, H, W, bt):
    M, Cin = x.shape
    HW = H * W
    p1, p2, p3 = p["conv1"], p["conv2"], p["conv3"]
    width = p1["w"].shape[-1]
    Cout = p3["w"].shape[-1]
    has_down = "down" in p
    rows = bt * HW

    args = [x, p1["w"][0], p1["scale"], p1["bias"],
            p2["w"], p2["scale"], p2["bias"],
            p3["w"][0], p3["scale"], p3["bias"]]
    in_specs = [pl.BlockSpec((rows, Cin), lambda i: (i, 0)),
                pl.BlockSpec((Cin, width), lambda i: (0, 0)),
                pl.BlockSpec((1, width), lambda i: (0, 0)),
                pl.BlockSpec((1, width), lambda i: (0, 0)),
                pl.BlockSpec((9, width, width), lambda i: (0, 0, 0)),
                pl.BlockSpec((1, width), lambda i: (0, 0)),
                pl.BlockSpec((1, width), lambda i: (0, 0)),
                pl.BlockSpec((width, Cout), lambda i: (0, 0)),
                pl.BlockSpec((1, Cout), lambda i: (0, 0)),
                pl.BlockSpec((1, Cout), lambda i: (0, 0))]
    if has_down:
        pd = p["down"]
        args += [pd["w"][0], pd["scale"], pd["bias"]]
        in_specs += [pl.BlockSpec((Cin, Cout), lambda i: (0, 0)),
                     pl.BlockSpec((1, Cout), lambda i: (0, 0)),
                     pl.BlockSpec((1, Cout), lambda i: (0, 0))]

    return pl.pallas_call(
        functools.partial(_bneck1_kernel, bt=bt, H=H, W=W, width=width,
                          has_down=has_down),
        grid=(M // rows,),
        in_specs=in_specs,
        out_specs=pl.BlockSpec((rows, Cout), lambda i: (i, 0)),
        out_shape=jax.ShapeDtypeStruct((M, Cout), jnp.bfloat16),
        scratch_shapes=[pltpu.VMEM((bt, H + 2, W + 2, width), jnp.bfloat16)],
        compiler_params=_cparams("parallel"),
    )(*args)


def _bneck2_kernel(x_ref, w1_ref, s1_ref, b1_ref, w2_ref, s2_ref, b2_ref,
                   w3_ref, s3_ref, b3_ref, wd_ref, sd_ref, bd_ref,
                   o_ref, pad_ref, *, bt, H, W, width):
    Ho, Wo = H // 2, W // 2
    Mo = bt * Ho * Wo
    Cin = x_ref.shape[-1]
    xb = x_ref[...]
    t1 = jnp.dot(xb, w1_ref[...], preferred_element_type=jnp.float32)
    t1 = jnp.maximum(t1 * s1_ref[...] + b1_ref[...], 0.0).astype(jnp.bfloat16)
    pad_ref[:, 0, :, :] = jnp.zeros((bt, W + 2, width), jnp.bfloat16)
    pad_ref[:, 1:H + 1, 0, :] = jnp.zeros((bt, H, width), jnp.bfloat16)
    pad_ref[:, 1:H + 1, 1:W + 1, :] = t1.reshape(bt, H, W, width)
    y6 = pad_ref[...].reshape(bt, (H + 2) // 2, 2, (W + 2) // 2, 2, width)
    acc = None
    for di in range(3):
        p, r0 = (di % 2, di // 2)
        for dj in range(3):
            q, c0 = (dj % 2, dj // 2)
            tap = y6[:, r0:r0 + Ho, p, c0:c0 + Wo, q, :]
            part = jnp.dot(tap.reshape(Mo, width), w2_ref[di * 3 + dj],
                           preferred_element_type=jnp.float32)
            acc = part if acc is None else acc + part
    t2 = jnp.maximum(acc * s2_ref[...] + b2_ref[...], 0.0).astype(jnp.bfloat16)
    out = jnp.dot(t2, w3_ref[...], preferred_element_type=jnp.float32)
    out = out * s3_ref[...] + b3_ref[...]
    xs = xb.reshape(bt, H // 2, 2, W // 2, 2, Cin)[:, :, 0, :, 0, :]
    xs = xs.reshape(Mo, Cin)
    ident = jnp.dot(xs, wd_ref[...], preferred_element_type=jnp.float32)
    ident = ident * sd_ref[...] + bd_ref[...]
    o_ref[...] = jnp.maximum(out + ident, 0.0).astype(o_ref.dtype)


def _bneck2(x, p, *, H, W, bt):
    M, Cin = x.shape
    B = M // (H * W)
    Ho, Wo = H // 2, W // 2
    p1, p2, p3, pd = p["conv1"], p["conv2"], p["conv3"], p["down"]
    width = p1["w"].shape[-1]
    Cout = p3["w"].shape[-1]
    rows_in = bt * H * W
    rows_out = bt * Ho * Wo

    in_specs = [pl.BlockSpec((rows_in, Cin), lambda i: (i, 0)),
                pl.BlockSpec((Cin, width), lambda i: (0, 0)),
                pl.BlockSpec((1, width), lambda i: (0, 0)),
                pl.BlockSpec((1, width), lambda i: (0, 0)),
                pl.BlockSpec((9, width, width), lambda i: (0, 0, 0)),
                pl.BlockSpec((1, width), lambda i: (0, 0)),
                pl.BlockSpec((1, width), lambda i: (0, 0)),
                pl.BlockSpec((width, Cout), lambda i: (0, 0)),
                pl.BlockSpec((1, Cout), lambda i: (0, 0)),
                pl.BlockSpec((1, Cout), lambda i: (0, 0)),
                pl.BlockSpec((Cin, Cout), lambda i: (0, 0)),
                pl.BlockSpec((1, Cout), lambda i: (0, 0)),
                pl.BlockSpec((1, Cout), lambda i: (0, 0))]
    return pl.pallas_call(
        functools.partial(_bneck2_kernel, bt=bt, H=H, W=W, width=width),
        grid=(B // bt,),
        in_specs=in_specs,
        out_specs=pl.BlockSpec((rows_out, Cout), lambda i: (i, 0)),
        out_shape=jax.ShapeDtypeStruct((B * Ho * Wo, Cout), jnp.bfloat16),
        scratch_shapes=[pltpu.VMEM((bt, H + 2, W + 2, width), jnp.bfloat16)],
        compiler_params=_cparams("parallel"),
    )(x, p1["w"][0], p1["scale"], p1["bias"],
      p2["w"], p2["scale"], p2["bias"],
      p3["w"][0], p3["scale"], p3["bias"],
      pd["w"][0], pd["scale"], pd["bias"])


def _embed_kernel(f_ref, w_ref, b_ref, o_ref, *, B, HW):
    f = f_ref[...].astype(jnp.float32).reshape(B, HW, f_ref.shape[-1])
    pooled = jnp.mean(f, axis=1).astype(jnp.bfloat16)
    o_ref[...] = jnp.dot(pooled, w_ref[...],
                         preferred_element_type=jnp.float32) + b_ref[...]


def _pool_embed(x, embed_w, embed_b, *, B, HW):
    M, C = x.shape
    E = embed_w.shape[1]
    te = 128 if E % 128 == 0 else E
    return pl.pallas_call(
        functools.partial(_embed_kernel, B=B, HW=HW),
        grid=(E // te,),
        in_specs=[pl.BlockSpec((M, C), lambda j: (0, 0)),
                  pl.BlockSpec((C, te), lambda j: (0, j)),
                  pl.BlockSpec((1, te), lambda j: (0, j))],
        out_specs=pl.BlockSpec((B, te), lambda j: (0, j)),
        out_shape=jax.ShapeDtypeStruct((B, E), jnp.float32),
        compiler_params=_cparams("parallel"),
    )(x, embed_w, embed_b)


def kernel(images, stem_w, stem_scale, stem_bias, L0_B0_conv1_w, L0_B0_conv1_scale, L0_B0_conv1_bias, L0_B0_conv2_w, L0_B0_conv2_scale, L0_B0_conv2_bias, L0_B0_conv3_w, L0_B0_conv3_scale, L0_B0_conv3_bias, L0_B0_down_w, L0_B0_down_scale, L0_B0_down_bias, L0_B1_conv1_w, L0_B1_conv1_scale, L0_B1_conv1_bias, L0_B1_conv2_w, L0_B1_conv2_scale, L0_B1_conv2_bias, L0_B1_conv3_w, L0_B1_conv3_scale, L0_B1_conv3_bias, L0_B2_conv1_w, L0_B2_conv1_scale, L0_B2_conv1_bias, L0_B2_conv2_w, L0_B2_conv2_scale, L0_B2_conv2_bias, L0_B2_conv3_w, L0_B2_conv3_scale, L0_B2_conv3_bias, L1_B0_conv1_w, L1_B0_conv1_scale, L1_B0_conv1_bias, L1_B0_conv2_w, L1_B0_conv2_scale, L1_B0_conv2_bias, L1_B0_conv3_w, L1_B0_conv3_scale, L1_B0_conv3_bias, L1_B0_down_w, L1_B0_down_scale, L1_B0_down_bias, L1_B1_conv1_w, L1_B1_conv1_scale, L1_B1_conv1_bias, L1_B1_conv2_w, L1_B1_conv2_scale, L1_B1_conv2_bias, L1_B1_conv3_w, L1_B1_conv3_scale, L1_B1_conv3_bias, L1_B2_conv1_w, L1_B2_conv1_scale, L1_B2_conv1_bias, L1_B2_conv2_w, L1_B2_conv2_scale, L1_B2_conv2_bias, L1_B2_conv3_w, L1_B2_conv3_scale, L1_B2_conv3_bias, L1_B3_conv1_w, L1_B3_conv1_scale, L1_B3_conv1_bias, L1_B3_conv2_w, L1_B3_conv2_scale, L1_B3_conv2_bias, L1_B3_conv3_w, L1_B3_conv3_scale, L1_B3_conv3_bias, L2_B0_conv1_w, L2_B0_conv1_scale, L2_B0_conv1_bias, L2_B0_conv2_w, L2_B0_conv2_scale, L2_B0_conv2_bias, L2_B0_conv3_w, L2_B0_conv3_scale, L2_B0_conv3_bias, L2_B0_down_w, L2_B0_down_scale, L2_B0_down_bias, L2_B1_conv1_w, L2_B1_conv1_scale, L2_B1_conv1_bias, L2_B1_conv2_w, L2_B1_conv2_scale, L2_B1_conv2_bias, L2_B1_conv3_w, L2_B1_conv3_scale, L2_B1_conv3_bias, L2_B2_conv1_w, L2_B2_conv1_scale, L2_B2_conv1_bias, L2_B2_conv2_w, L2_B2_conv2_scale, L2_B2_conv2_bias, L2_B2_conv3_w, L2_B2_conv3_scale, L2_B2_conv3_bias, L2_B3_conv1_w, L2_B3_conv1_scale, L2_B3_conv1_bias, L2_B3_conv2_w, L2_B3_conv2_scale, L2_B3_conv2_bias, L2_B3_conv3_w, L2_B3_conv3_scale, L2_B3_conv3_bias, L2_B4_conv1_w, L2_B4_conv1_scale, L2_B4_conv1_bias, L2_B4_conv2_w, L2_B4_conv2_scale, L2_B4_conv2_bias, L2_B4_conv3_w, L2_B4_conv3_scale, L2_B4_conv3_bias, L2_B5_conv1_w, L2_B5_conv1_scale, L2_B5_conv1_bias, L2_B5_conv2_w, L2_B5_conv2_scale, L2_B5_conv2_bias, L2_B5_conv3_w, L2_B5_conv3_scale, L2_B5_conv3_bias, L3_B0_conv1_w, L3_B0_conv1_scale, L3_B0_conv1_bias, L3_B0_conv2_w, L3_B0_conv2_scale, L3_B0_conv2_bias, L3_B0_conv3_w, L3_B0_conv3_scale, L3_B0_conv3_bias, L3_B0_down_w, L3_B0_down_scale, L3_B0_down_bias, L3_B1_conv1_w, L3_B1_conv1_scale, L3_B1_conv1_bias, L3_B1_conv2_w, L3_B1_conv2_scale, L3_B1_conv2_bias, L3_B1_conv3_w, L3_B1_conv3_scale, L3_B1_conv3_bias, L3_B2_conv1_w, L3_B2_conv1_scale, L3_B2_conv1_bias, L3_B2_conv2_w, L3_B2_conv2_scale, L3_B2_conv2_bias, L3_B2_conv3_w, L3_B2_conv3_scale, L3_B2_conv3_bias, embed_w, embed_b):
    ns = locals()

    def blk(li, bi, stride):
        d = {}
        for cname in ("conv1", "conv2", "conv3", "down"):
            key = "L%d_B%d_%s_w" % (li, bi, cname)
            if key in ns:
                d[cname] = {"w": ns[key],
                            "scale": ns["L%d_B%d_%s_scale" % (li, bi, cname)],
                            "bias": ns["L%d_B%d_%s_bias" % (li, bi, cname)]}
        d["stride"] = stride
        return d

    B = images.shape[0]
    x = _stem(images, stem_w, stem_scale, stem_bias)

    x = _bneck1(x, blk(0, 0, 1), H=56, W=56, bt=1)
    x = _bneck1(x, blk(0, 1, 1), H=56, W=56, bt=1)
    x = _bneck1(x, blk(0, 2, 1), H=56, W=56, bt=1)

    x = _bneck2(x, blk(1, 0, 2), H=56, W=56, bt=2)
    x = _bneck1(x, blk(1, 1, 1), H=28, W=28, bt=2)
    x = _bneck1(x, blk(1, 2, 1), H=28, W=28, bt=2)
    x = _bneck1(x, blk(1, 3, 1), H=28, W=28, bt=2)

    x = _bneck2(x, blk(2, 0, 2), H=28, W=28, bt=4)
    x = _bneck1(x, blk(2, 1, 1), H=14, W=14, bt=4)
    x = _bneck1(x, blk(2, 2, 1), H=14, W=14, bt=4)
    x = _bneck1(x, blk(2, 3, 1), H=14, W=14, bt=4)
    x = _bneck1(x, blk(2, 4, 1), H=14, W=14, bt=4)
    x = _bneck1(x, blk(2, 5, 1), H=14, W=14, bt=4)

    x = _bneck2(x, blk(3, 0, 2), H=14, W=14, bt=8)
    x = _bneck1(x, blk(3, 1, 1), H=7, W=7, bt=8)
    x = _bneck1(x, blk(3, 2, 1), H=7, W=7, bt=8)

    return _pool_embed(x, embed_w, embed_b, B=B, HW=49)


# speedup vs baseline: 2.5313x; 1.0390x over previous
import functools

import jax
import jax.numpy as jnp
from jax.experimental import pallas as pl
from jax.experimental.pallas import tpu as pltpu

_VMEM_LIMIT = 56 * 1024 * 1024


def _cparams(*sem):
    return pltpu.CompilerParams(dimension_semantics=sem,
                                vmem_limit_bytes=_VMEM_LIMIT)


_STEM_SLABS = [(p, s, c)
               for p in range(2)
               for s in (range(-1, 2) if p == 0 else range(-2, 2))
               for c in range(3)]


def _stem_kernel(x_ref, t_ref, s_ref, b_ref, o_ref):
    xi = x_ref[:, 0].astype(jnp.bfloat16).reshape(2, 3, 112, 2, 64)
    z2 = jnp.zeros((2, 64), jnp.bfloat16)
    z1 = jnp.zeros((1, 64), jnp.bfloat16)
    padded = {}
    for g in range(2):
        for c in range(3):
            for p in range(2):
                padded[(g, p, c)] = jnp.concatenate(
                    [z2, xi[g, c, :, p, :], z1], axis=0)
    acc = None
    for idx, (p, s, c) in enumerate(_STEM_SLABS):
        lhs = jnp.concatenate([padded[(0, p, c)][s + 2:s + 114],
                               padded[(1, p, c)][s + 2:s + 114]], axis=0)
        part = jnp.dot(lhs, t_ref[idx], preferred_element_type=jnp.float32)
        acc = part if acc is None else acc + part
    y = jnp.maximum(acc * s_ref[...] + b_ref[...], 0.0)
    y = y.astype(jnp.bfloat16)
    y4 = y.reshape(2, 56, 2, 1856)
    e, o = y4[:, :, 0], y4[:, :, 1]
    pz = jnp.zeros((2, 1, 1856), jnp.bfloat16)
    prev = jnp.concatenate([pz, o[:, :-1]], axis=1)
    o_ref[:, 0] = jnp.maximum(jnp.maximum(e, o), prev)


def _colpool_kernel(x_ref, o_ref):
    x4 = x_ref[0]
    outs = []
    for m in range(4):
        x = x4[m]
        if m == 0:
            jj = jax.lax.broadcasted_iota(jnp.int32, (1, 29, 1), 1)
            x = jnp.where(jj == 0, jnp.bfloat16(0), x)
        e4 = x[:, 0:28].reshape(56, 14, 2, 64)
        ce, co = e4[:, :, 0], e4[:, :, 1]
        z = x[:, 1:29].reshape(56, 14, 2, 64)[:, :, 1]
        outs.append(jnp.maximum(jnp.maximum(ce, co), z)[:, None])
    pooled = jnp.concatenate(outs, axis=1)
    o_ref[0] = pooled.reshape(3136, 64)


def _stem(images, stem_w, stem_scale, stem_bias):
    B = images.shape[0]
    imgp = jnp.pad(images, ((0, 0), (0, 0), (0, 0), (5, 3)))
    img4 = jnp.stack([imgp[:, :, :, 56 * m:56 * m + 64] for m in range(4)],
                     axis=1)
    w = stem_w.astype(jnp.bfloat16)
    k = jnp.arange(64)[:, None]
    jj = jnp.arange(29)[None, :]
    hots = [(k == 2 * jj + dj).astype(jnp.bfloat16) for dj in range(7)]
    tws = []
    for (p, s, c) in _STEM_SLABS:
        di = 2 * s + 3 + p
        t = sum(hots[dj][:, :, None] * w[di * 7 + dj, c][None, None, :]
                for dj in range(7))
        tws.append(t.reshape(64, 1856))
    tw = jnp.stack(tws)
    sc = jnp.tile(stem_scale, (1, 29))
    bi = jnp.tile(stem_bias, (1, 29))
    ym = pl.pallas_call(
        _stem_kernel,
        grid=(4, B // 2),
        in_specs=[pl.BlockSpec((2, 1, 3, 224, 64), lambda m, b: (b, m, 0, 0, 0)),
                  pl.BlockSpec((21, 64, 1856), lambda m, b: (0, 0, 0)),
                  pl.BlockSpec((1, 1856), lambda m, b: (0, 0)),
                  pl.BlockSpec((1, 1856), lambda m, b: (0, 0))],
        out_specs=pl.BlockSpec((2, 1, 56, 1856), lambda m, b: (b, m, 0, 0)),
        out_shape=jax.ShapeDtypeStruct((B, 4, 56, 1856), jnp.bfloat16),
        compiler_params=_cparams("parallel", "parallel"),
    )(img4, tw, sc, bi)
    ym = ym.reshape(B, 4, 56, 29, 64)
    out = pl.pallas_call(
        _colpool_kernel,
        grid=(B,),
        in_specs=[pl.BlockSpec((1, 4, 56, 29, 64),
                               lambda b: (b, 0, 0, 0, 0))],
        out_specs=pl.BlockSpec((1, 3136, 64), lambda b: (b, 0, 0)),
        out_shape=jax.ShapeDtypeStruct((B, 3136, 64), jnp.bfloat16),
        compiler_params=_cparams("parallel"),
    )(ym)
    return out.reshape(B * 3136, 64)


def _bneck1_kernel(x_ref, w1_ref, s1_ref, b1_ref, w2_ref, s2_ref, b2_ref,
                   w3_ref, s3_ref, b3_ref, *rest, bt, H, W, width, has_down):
    if has_down:
        wd_ref, sd_ref, bd_ref, o_ref, pad_ref = rest
    else:
        o_ref, pad_ref = rest
    HW = H * W
    xb = x_ref[...]
    t1 = jnp.dot(xb, w1_ref[...], preferred_element_type=jnp.float32)
    t1 = jnp.maximum(t1 * s1_ref[...] + b1_ref[...], 0.0).astype(jnp.bfloat16)
    pad_ref[:, 0, :, :] = jnp.zeros((bt, W + 2, width), jnp.bfloat16)
    pad_ref[:, H + 1, :, :] = jnp.zeros((bt, W + 2, width), jnp.bfloat16)
    pad_ref[:, 1:H + 1, 0, :] = jnp.zeros((bt, H, width), jnp.bfloat16)
    pad_ref[:, 1:H + 1, W + 1, :] = jnp.zeros((bt, H, width), jnp.bfloat16)
    pad_ref[:, 1:H + 1, 1:W + 1, :] = t1.reshape(bt, H, W, width)
    padded = pad_ref[...]
    acc = None
    for di in range(3):
        for dj in range(3):
            tap = padded[:, di:di + H, dj:dj + W, :].reshape(bt * HW, width)
            part = jnp.dot(tap, w2_ref[di * 3 + dj],
                           preferred_element_type=jnp.float32)
            acc = part if acc is None else acc + part
    t2 = jnp.maximum(acc * s2_ref[...] + b2_ref[...], 0.0).astype(jnp.bfloat16)
    out = jnp.dot(t2, w3_ref[...], preferred_element_type=jnp.float32)
    out = out * s3_ref[...] + b3_ref[...]
    if has_down:
        ident = jnp.dot(xb, wd_ref[...], preferred_element_type=jnp.float32)
        ident = ident * sd_ref[...] + bd_ref[...]
    else:
        ident = xb.astype(jnp.float32)
    o_ref[...] = jnp.maximum(out + ident, 0.0).astype(o_ref.dtype)


def _bneck1(x, p, *, H, W, bt):
    M, Cin = x.shape
    HW = H * W
    p1, p2, p3 = p["conv1"], p["conv2"], p["conv3"]
    width = p1["w"].shape[-1]
    Cout = p3["w"].shape[-1]
    has_down = "down" in p
    rows = bt * HW

    args = [x, p1["w"][0], p1["scale"], p1["bias"],
            p2["w"], p2["scale"], p2["bias"],
            p3["w"][0], p3["scale"], p3["bias"]]
    in_specs = [pl.BlockSpec((rows, Cin), lambda i: (i, 0)),
                pl.BlockSpec((Cin, width), lambda i: (0, 0)),
                pl.BlockSpec((1, width), lambda i: (0, 0)),
                pl.BlockSpec((1, width), lambda i: (0, 0)),
                pl.BlockSpec((9, width, width), lambda i: (0, 0, 0)),
                pl.BlockSpec((1, width), lambda i: (0, 0)),
                pl.BlockSpec((1, width), lambda i: (0, 0)),
                pl.BlockSpec((width, Cout), lambda i: (0, 0)),
                pl.BlockSpec((1, Cout), lambda i: (0, 0)),
                pl.BlockSpec((1, Cout), lambda i: (0, 0))]
    if has_down:
        pd = p["down"]
        args += [pd["w"][0], pd["scale"], pd["bias"]]
        in_specs += [pl.BlockSpec((Cin, Cout), lambda i: (0, 0)),
                     pl.BlockSpec((1, Cout), lambda i: (0, 0)),
                     pl.BlockSpec((1, Cout), lambda i: (0, 0))]

    return pl.pallas_call(
        functools.partial(_bneck1_kernel, bt=bt, H=H, W=W, width=width,
                          has_down=has_down),
        grid=(M // rows,),
        in_specs=in_specs,
        out_specs=pl.BlockSpec((rows, Cout), lambda i: (i, 0)),
        out_shape=jax.ShapeDtypeStruct((M, Cout), jnp.bfloat16),
        scratch_shapes=[pltpu.VMEM((bt, H + 2, W + 2, width), jnp.bfloat16)],
        compiler_params=_cparams("parallel"),
    )(*args)


def _bneck2_kernel(x_ref, w1_ref, s1_ref, b1_ref, w2_ref, s2_ref, b2_ref,
                   w3_ref, s3_ref, b3_ref, wd_ref, sd_ref, bd_ref,
                   o_ref, pad_ref, *, bt, H, W, width):
    Ho, Wo = H // 2, W // 2
    Mo = bt * Ho * Wo
    Cin = x_ref.shape[-1]
    xb = x_ref[...]
    t1 = jnp.dot(xb, w1_ref[...], preferred_element_type=jnp.float32)
    t1 = jnp.maximum(t1 * s1_ref[...] + b1_ref[...], 0.0).astype(jnp.bfloat16)
    pad_ref[:, 0, :, :] = jnp.zeros((bt, W + 2, width), jnp.bfloat16)
    pad_ref[:, 1:H + 1, 0, :] = jnp.zeros((bt, H, width), jnp.bfloat16)
    pad_ref[:, 1:H + 1, 1:W + 1, :] = t1.reshape(bt, H, W, width)
    y6 = pad_ref[...].reshape(bt, (H + 2) // 2, 2, (W + 2) // 2, 2, width)
    acc = None
    for di in range(3):
        p, r0 = (di % 2, di // 2)
        for dj in range(3):
            q, c0 = (dj % 2, dj // 2)
            tap = y6[:, r0:r0 + Ho, p, c0:c0 + Wo, q, :]
            part = jnp.dot(tap.reshape(Mo, width), w2_ref[di * 3 + dj],
                           preferred_element_type=jnp.float32)
            acc = part if acc is None else acc + part
    t2 = jnp.maximum(acc * s2_ref[...] + b2_ref[...], 0.0).astype(jnp.bfloat16)
    out = jnp.dot(t2, w3_ref[...], preferred_element_type=jnp.float32)
    out = out * s3_ref[...] + b3_ref[...]
    xs = xb.reshape(bt, H // 2, 2, W // 2, 2, Cin)[:, :, 0, :, 0, :]
    xs = xs.reshape(Mo, Cin)
    ident = jnp.dot(xs, wd_ref[...], preferred_element_type=jnp.float32)
    ident = ident * sd_ref[...] + bd_ref[...]
    o_ref[...] = jnp.maximum(out + ident, 0.0).astype(o_ref.dtype)


def _bneck2(x, p, *, H, W, bt):
    M, Cin = x.shape
    B = M // (H * W)
    Ho, Wo = H // 2, W // 2
    p1, p2, p3, pd = p["conv1"], p["conv2"], p["conv3"], p["down"]
    width = p1["w"].shape[-1]
    Cout = p3["w"].shape[-1]
    rows_in = bt * H * W
    rows_out = bt * Ho * Wo

    in_specs = [pl.BlockSpec((rows_in, Cin), lambda i: (i, 0)),
                pl.BlockSpec((Cin, width), lambda i: (0, 0)),
                pl.BlockSpec((1, width), lambda i: (0, 0)),
                pl.BlockSpec((1, width), lambda i: (0, 0)),
                pl.BlockSpec((9, width, width), lambda i: (0, 0, 0)),
                pl.BlockSpec((1, width), lambda i: (0, 0)),
                pl.BlockSpec((1, width), lambda i: (0, 0)),
                pl.BlockSpec((width, Cout), lambda i: (0, 0)),
                pl.BlockSpec((1, Cout), lambda i: (0, 0)),
                pl.BlockSpec((1, Cout), lambda i: (0, 0)),
                pl.BlockSpec((Cin, Cout), lambda i: (0, 0)),
                pl.BlockSpec((1, Cout), lambda i: (0, 0)),
                pl.BlockSpec((1, Cout), lambda i: (0, 0))]
    return pl.pallas_call(
        functools.partial(_bneck2_kernel, bt=bt, H=H, W=W, width=width),
        grid=(B // bt,),
        in_specs=in_specs,
        out_specs=pl.BlockSpec((rows_out, Cout), lambda i: (i, 0)),
        out_shape=jax.ShapeDtypeStruct((B * Ho * Wo, Cout), jnp.bfloat16),
        scratch_shapes=[pltpu.VMEM((bt, H + 2, W + 2, width), jnp.bfloat16)],
        compiler_params=_cparams("parallel"),
    )(x, p1["w"][0], p1["scale"], p1["bias"],
      p2["w"], p2["scale"], p2["bias"],
      p3["w"][0], p3["scale"], p3["bias"],
      pd["w"][0], pd["scale"], pd["bias"])


def _embed_kernel(f_ref, w_ref, b_ref, o_ref, *, B, HW):
    f = f_ref[...].astype(jnp.float32).reshape(B, HW, f_ref.shape[-1])
    pooled = jnp.mean(f, axis=1).astype(jnp.bfloat16)
    o_ref[...] = jnp.dot(pooled, w_ref[...],
                         preferred_element_type=jnp.float32) + b_ref[...]


def _pool_embed(x, embed_w, embed_b, *, B, HW):
    M, C = x.shape
    E = embed_w.shape[1]
    te = 128 if E % 128 == 0 else E
    return pl.pallas_call(
        functools.partial(_embed_kernel, B=B, HW=HW),
        grid=(E // te,),
        in_specs=[pl.BlockSpec((M, C), lambda j: (0, 0)),
                  pl.BlockSpec((C, te), lambda j: (0, j)),
                  pl.BlockSpec((1, te), lambda j: (0, j))],
        out_specs=pl.BlockSpec((B, te), lambda j: (0, j)),
        out_shape=jax.ShapeDtypeStruct((B, E), jnp.float32),
        compiler_params=_cparams("parallel"),
    )(x, embed_w, embed_b)


def kernel(images, stem_w, stem_scale, stem_bias, L0_B0_conv1_w, L0_B0_conv1_scale, L0_B0_conv1_bias, L0_B0_conv2_w, L0_B0_conv2_scale, L0_B0_conv2_bias, L0_B0_conv3_w, L0_B0_conv3_scale, L0_B0_conv3_bias, L0_B0_down_w, L0_B0_down_scale, L0_B0_down_bias, L0_B1_conv1_w, L0_B1_conv1_scale, L0_B1_conv1_bias, L0_B1_conv2_w, L0_B1_conv2_scale, L0_B1_conv2_bias, L0_B1_conv3_w, L0_B1_conv3_scale, L0_B1_conv3_bias, L0_B2_conv1_w, L0_B2_conv1_scale, L0_B2_conv1_bias, L0_B2_conv2_w, L0_B2_conv2_scale, L0_B2_conv2_bias, L0_B2_conv3_w, L0_B2_conv3_scale, L0_B2_conv3_bias, L1_B0_conv1_w, L1_B0_conv1_scale, L1_B0_conv1_bias, L1_B0_conv2_w, L1_B0_conv2_scale, L1_B0_conv2_bias, L1_B0_conv3_w, L1_B0_conv3_scale, L1_B0_conv3_bias, L1_B0_down_w, L1_B0_down_scale, L1_B0_down_bias, L1_B1_conv1_w, L1_B1_conv1_scale, L1_B1_conv1_bias, L1_B1_conv2_w, L1_B1_conv2_scale, L1_B1_conv2_bias, L1_B1_conv3_w, L1_B1_conv3_scale, L1_B1_conv3_bias, L1_B2_conv1_w, L1_B2_conv1_scale, L1_B2_conv1_bias, L1_B2_conv2_w, L1_B2_conv2_scale, L1_B2_conv2_bias, L1_B2_conv3_w, L1_B2_conv3_scale, L1_B2_conv3_bias, L1_B3_conv1_w, L1_B3_conv1_scale, L1_B3_conv1_bias, L1_B3_conv2_w, L1_B3_conv2_scale, L1_B3_conv2_bias, L1_B3_conv3_w, L1_B3_conv3_scale, L1_B3_conv3_bias, L2_B0_conv1_w, L2_B0_conv1_scale, L2_B0_conv1_bias, L2_B0_conv2_w, L2_B0_conv2_scale, L2_B0_conv2_bias, L2_B0_conv3_w, L2_B0_conv3_scale, L2_B0_conv3_bias, L2_B0_down_w, L2_B0_down_scale, L2_B0_down_bias, L2_B1_conv1_w, L2_B1_conv1_scale, L2_B1_conv1_bias, L2_B1_conv2_w, L2_B1_conv2_scale, L2_B1_conv2_bias, L2_B1_conv3_w, L2_B1_conv3_scale, L2_B1_conv3_bias, L2_B2_conv1_w, L2_B2_conv1_scale, L2_B2_conv1_bias, L2_B2_conv2_w, L2_B2_conv2_scale, L2_B2_conv2_bias, L2_B2_conv3_w, L2_B2_conv3_scale, L2_B2_conv3_bias, L2_B3_conv1_w, L2_B3_conv1_scale, L2_B3_conv1_bias, L2_B3_conv2_w, L2_B3_conv2_scale, L2_B3_conv2_bias, L2_B3_conv3_w, L2_B3_conv3_scale, L2_B3_conv3_bias, L2_B4_conv1_w, L2_B4_conv1_scale, L2_B4_conv1_bias, L2_B4_conv2_w, L2_B4_conv2_scale, L2_B4_conv2_bias, L2_B4_conv3_w, L2_B4_conv3_scale, L2_B4_conv3_bias, L2_B5_conv1_w, L2_B5_conv1_scale, L2_B5_conv1_bias, L2_B5_conv2_w, L2_B5_conv2_scale, L2_B5_conv2_bias, L2_B5_conv3_w, L2_B5_conv3_scale, L2_B5_conv3_bias, L3_B0_conv1_w, L3_B0_conv1_scale, L3_B0_conv1_bias, L3_B0_conv2_w, L3_B0_conv2_scale, L3_B0_conv2_bias, L3_B0_conv3_w, L3_B0_conv3_scale, L3_B0_conv3_bias, L3_B0_down_w, L3_B0_down_scale, L3_B0_down_bias, L3_B1_conv1_w, L3_B1_conv1_scale, L3_B1_conv1_bias, L3_B1_conv2_w, L3_B1_conv2_scale, L3_B1_conv2_bias, L3_B1_conv3_w, L3_B1_conv3_scale, L3_B1_conv3_bias, L3_B2_conv1_w, L3_B2_conv1_scale, L3_B2_conv1_bias, L3_B2_conv2_w, L3_B2_conv2_scale, L3_B2_conv2_bias, L3_B2_conv3_w, L3_B2_conv3_scale, L3_B2_conv3_bias, embed_w, embed_b):
    ns = locals()

    def blk(li, bi, stride):
        d = {}
        for cname in ("conv1", "conv2", "conv3", "down"):
            key = "L%d_B%d_%s_w" % (li, bi, cname)
            if key in ns:
                d[cname] = {"w": ns[key],
                            "scale": ns["L%d_B%d_%s_scale" % (li, bi, cname)],
                            "bias": ns["L%d_B%d_%s_bias" % (li, bi, cname)]}
        d["stride"] = stride
        return d

    B = images.shape[0]
    x = _stem(images, stem_w, stem_scale, stem_bias)

    x = _bneck1(x, blk(0, 0, 1), H=56, W=56, bt=1)
    x = _bneck1(x, blk(0, 1, 1), H=56, W=56, bt=1)
    x = _bneck1(x, blk(0, 2, 1), H=56, W=56, bt=1)

    x = _bneck2(x, blk(1, 0, 2), H=56, W=56, bt=2)
    x = _bneck1(x, blk(1, 1, 1), H=28, W=28, bt=2)
    x = _bneck1(x, blk(1, 2, 1), H=28, W=28, bt=2)
    x = _bneck1(x, blk(1, 3, 1), H=28, W=28, bt=2)

    x = _bneck2(x, blk(2, 0, 2), H=28, W=28, bt=4)
    x = _bneck1(x, blk(2, 1, 1), H=14, W=14, bt=4)
    x = _bneck1(x, blk(2, 2, 1), H=14, W=14, bt=4)
    x = _bneck1(x, blk(2, 3, 1), H=14, W=14, bt=4)
    x = _bneck1(x, blk(2, 4, 1), H=14, W=14, bt=4)
    x = _bneck1(x, blk(2, 5, 1), H=14, W=14, bt=4)

    x = _bneck2(x, blk(3, 0, 2), H=14, W=14, bt=8)
    x = _bneck1(x, blk(3, 1, 1), H=7, W=7, bt=8)
    x = _bneck1(x, blk(3, 2, 1), H=7, W=7, bt=8)

    return _pool_embed(x, embed_w, embed_b, B=B, HW=49)
